# Initial kernel scaffold; baseline (speedup 1.0000x reference)
import jax
import jax.numpy as jnp
from jax.experimental import pallas as pl


def kernel(f_atoms, f_bonds, a2b, b2a, b2revb, a_scope, b_scope, a2a, features_batch, W_i, W_h1, W_h2, W_o):
    raise NotImplementedError("write your pallas kernel here")


# speedup vs baseline: 1.0414x; 1.0414x over previous
import functools

import jax
import jax.numpy as jnp
from jax import lax
from jax.experimental import pallas as pl
from jax.experimental.pallas import tpu as pltpu
from jax.experimental.pallas import tpu_sc as plsc

N, D, DEG = 10000, 128, 32
NW = 32
APW = 320
NPAD = NW * APW
CH = 4
ROWS = CH * DEG
NCH = APW // CH
LG = D // 16


def _gs_body(table_hbm, idx_hbm, out_hbm, idx_v, rows_v, out_v, sem):
    wid = lax.axis_index("s") * 2 + lax.axis_index("c")
    pltpu.sync_copy(idx_hbm.at[wid], idx_v)

    def process(buf, c):
        for a in range(CH):
            def jbody(j, accs, a=a):
                return tuple(accs[g] + buf[a * DEG + j, pl.ds(g * 16, 16)]
                             for g in range(LG))
            accs = lax.fori_loop(
                0, DEG, jbody,
                tuple(jnp.zeros((16,), jnp.float32) for _ in range(LG)))
            for g in range(LG):
                out_v[c * CH + a, pl.ds(g * 16, 16)] = accs[g]

    def cbody(c, carry):
        pltpu.async_copy(table_hbm.at[idx_v.at[c]], rows_v, sem).wait()
        process(rows_v, c)
        return carry

    lax.fori_loop(0, NCH, cbody, 0)
    pltpu.sync_copy(out_v, out_hbm.at[pl.ds(wid * APW, APW)])


def _gather_sum(table, idx3):
    gs = functools.partial(
        pl.kernel,
        out_type=jax.ShapeDtypeStruct((NPAD, D), jnp.float32),
        mesh=plsc.VectorSubcoreMesh(core_axis_name="c", subcore_axis_name="s"),
        scratch_types=[
            pltpu.VMEM((NCH, ROWS), jnp.int32),
            pltpu.VMEM((ROWS, D), jnp.float32),
            pltpu.VMEM((APW, D), jnp.float32),
            pltpu.SemaphoreType.DMA,
        ],
    )(_gs_body)
    return gs(table, idx3)


def _prep_idx(a2x):
    flat = a2x.astype(jnp.int32).reshape(-1)
    flat = jnp.pad(flat, (0, (NPAD - N) * DEG))
    return flat.reshape(NW, NCH, ROWS)


def _mm2_relu_body(a_ref, b_ref, wa_ref, wb_ref, o_ref):
    acc = jnp.dot(a_ref[...], wa_ref[...], preferred_element_type=jnp.float32)
    acc = acc + jnp.dot(b_ref[...], wb_ref[...], preferred_element_type=jnp.float32)
    o_ref[...] = jnp.maximum(acc, 0.0)


def _mm2_relu(a, b, wa, wb):
    n = a.shape[0]
    blk = 2000
    return pl.pallas_call(
        _mm2_relu_body,
        grid=(n // blk,),
        in_specs=[
            pl.BlockSpec((blk, D), lambda i: (i, 0)),
            pl.BlockSpec((blk, D), lambda i: (i, 0)),
            pl.BlockSpec((D, D), lambda i: (0, 0)),
            pl.BlockSpec((D, D), lambda i: (0, 0)),
        ],
        out_specs=pl.BlockSpec((blk, D), lambda i: (i, 0)),
        out_shape=jax.ShapeDtypeStruct((n, D), jnp.float32),
    )(a, b, wa, wb)


def _res_mm_relu_body(x_ref, g_ref, w_ref, o_ref):
    acc = jnp.dot(g_ref[...], w_ref[...], preferred_element_type=jnp.float32)
    o_ref[...] = jnp.maximum(x_ref[...] + acc, 0.0)


def _res_mm_relu(x, g, w):
    n = x.shape[0]
    blk = 2000
    return pl.pallas_call(
        _res_mm_relu_body,
        grid=(n // blk,),
        in_specs=[
            pl.BlockSpec((blk, D), lambda i: (i, 0)),
            pl.BlockSpec((blk, D), lambda i: (i, 0)),
            pl.BlockSpec((D, D), lambda i: (0, 0)),
        ],
        out_specs=pl.BlockSpec((blk, D), lambda i: (i, 0)),
        out_shape=jax.ShapeDtypeStruct((n, D), jnp.float32),
    )(x, g, w)


def kernel(f_atoms, f_bonds, a2b, b2a, b2revb, a_scope, b_scope, a2a,
           features_batch, W_i, W_h1, W_h2, W_o):
    idx_b = _prep_idx(a2b)
    idx_a = _prep_idx(a2a)

    bond_agg = _gather_sum(f_bonds, idx_b)[:N]
    input_atom = _mm2_relu(f_atoms, bond_agg, W_i[:D], W_i[D:])
    message = input_atom
    for W_h in (W_h1, W_h2):
        agg = _gather_sum(message, idx_a)[:N]
        message = _res_mm_relu(input_atom, agg, W_h)
    return _mm2_relu(f_atoms, message, W_o[:D], W_o[D:])


# speedup vs baseline: 1.1858x; 1.1387x over previous
import functools

import jax
import jax.numpy as jnp
from jax import lax
from jax.experimental import pallas as pl
from jax.experimental.pallas import tpu as pltpu
from jax.experimental.pallas import tpu_sc as plsc

N, D, DEG = 10000, 128, 32
NW = 32
APW = 320
NPAD = NW * APW
CH = 4
ROWS = CH * DEG
NCH = APW // CH
LG = D // 16


NBUF = 4


def _gs_body(table_hbm, idx_hbm, out_hbm, idx_v, rows_bufs, out_v, sems):
    wid = lax.axis_index("s") * 2 + lax.axis_index("c")
    pltpu.sync_copy(idx_hbm.at[wid], idx_v)

    def process(buf, c):
        for a in range(CH):
            def jbody(j, accs, a=a):
                return tuple(accs[g] + buf[a * DEG + j, pl.ds(g * 16, 16)]
                             for g in range(LG))
            accs = lax.fori_loop(
                0, DEG, jbody,
                tuple(jnp.zeros((16,), jnp.float32) for _ in range(LG)))
            for g in range(LG):
                out_v[c * CH + a, pl.ds(g * 16, 16)] = accs[g]

    for b in range(NBUF):
        pltpu.async_copy(table_hbm.at[idx_v.at[b]], rows_bufs[b], sems[b])

    def cbody(i, carry):
        k = i * NBUF
        for b in range(NBUF):
            c = k + b
            pltpu.make_async_copy(
                table_hbm.at[idx_v.at[c]], rows_bufs[b], sems[b]).wait()
            process(rows_bufs[b], c)

            @pl.when(c + NBUF < NCH)
            def _(c=c, b=b):
                pltpu.async_copy(
                    table_hbm.at[idx_v.at[c + NBUF]], rows_bufs[b], sems[b])
        return carry

    lax.fori_loop(0, NCH // NBUF, cbody, 0)
    pltpu.sync_copy(out_v, out_hbm.at[pl.ds(wid * APW, APW)])


def _gs_entry(table_hbm, idx_hbm, out_hbm, idx_v, r0, r1, r2, r3, out_v,
              s0, s1, s2, s3):
    _gs_body(table_hbm, idx_hbm, out_hbm, idx_v, (r0, r1, r2, r3), out_v,
             (s0, s1, s2, s3))


def _gather_sum(table, idx3):
    gs = functools.partial(
        pl.kernel,
        out_type=jax.ShapeDtypeStruct((NPAD, D), jnp.float32),
        mesh=plsc.VectorSubcoreMesh(core_axis_name="c", subcore_axis_name="s"),
        scratch_types=[
            pltpu.VMEM((NCH, ROWS), jnp.int32),
        ] + [pltpu.VMEM((ROWS, D), jnp.float32) for _ in range(NBUF)] + [
            pltpu.VMEM((APW, D), jnp.float32),
        ] + [pltpu.SemaphoreType.DMA for _ in range(NBUF)],
    )(_gs_entry)
    return gs(table, idx3)


def _prep_idx(a2x):
    flat = a2x.astype(jnp.int32).reshape(-1)
    flat = jnp.pad(flat, (0, (NPAD - N) * DEG))
    return flat.reshape(NW, NCH, ROWS)


def _mm2_relu_body(a_ref, b_ref, wa_ref, wb_ref, o_ref):
    acc = jnp.dot(a_ref[...], wa_ref[...], preferred_element_type=jnp.float32)
    acc = acc + jnp.dot(b_ref[...], wb_ref[...], preferred_element_type=jnp.float32)
    o_ref[...] = jnp.maximum(acc, 0.0)


def _mm2_relu(a, b, wa, wb):
    n = a.shape[0]
    blk = 2000
    return pl.pallas_call(
        _mm2_relu_body,
        grid=(n // blk,),
        in_specs=[
            pl.BlockSpec((blk, D), lambda i: (i, 0)),
            pl.BlockSpec((blk, D), lambda i: (i, 0)),
            pl.BlockSpec((D, D), lambda i: (0, 0)),
            pl.BlockSpec((D, D), lambda i: (0, 0)),
        ],
        out_specs=pl.BlockSpec((blk, D), lambda i: (i, 0)),
        out_shape=jax.ShapeDtypeStruct((n, D), jnp.float32),
    )(a, b, wa, wb)


def _res_mm_relu_body(x_ref, g_ref, w_ref, o_ref):
    acc = jnp.dot(g_ref[...], w_ref[...], preferred_element_type=jnp.float32)
    o_ref[...] = jnp.maximum(x_ref[...] + acc, 0.0)


def _res_mm_relu(x, g, w):
    n = x.shape[0]
    blk = 2000
    return pl.pallas_call(
        _res_mm_relu_body,
        grid=(n // blk,),
        in_specs=[
            pl.BlockSpec((blk, D), lambda i: (i, 0)),
            pl.BlockSpec((blk, D), lambda i: (i, 0)),
            pl.BlockSpec((D, D), lambda i: (0, 0)),
        ],
        out_specs=pl.BlockSpec((blk, D), lambda i: (i, 0)),
        out_shape=jax.ShapeDtypeStruct((n, D), jnp.float32),
    )(x, g, w)


def kernel(f_atoms, f_bonds, a2b, b2a, b2revb, a_scope, b_scope, a2a,
           features_batch, W_i, W_h1, W_h2, W_o):
    idx_b = _prep_idx(a2b)
    idx_a = _prep_idx(a2a)

    bond_agg = _gather_sum(f_bonds, idx_b)[:N]
    input_atom = _mm2_relu(f_atoms, bond_agg, W_i[:D], W_i[D:])
    message = input_atom
    for W_h in (W_h1, W_h2):
        agg = _gather_sum(message, idx_a)[:N]
        message = _res_mm_relu(input_atom, agg, W_h)
    return _mm2_relu(f_atoms, message, W_o[:D], W_o[D:])
